# baseline (speedup 1.0000x reference)
import jax
import jax.numpy as jnp
from jax.experimental import pallas as pl
from jax.experimental.pallas import tpu as pltpu

N_B, N_TOKENS, N_T, N_C = 8, 675, 16, 256
N_BOT = 135
T_BLK = 8


_SQRT_HALF = 0.7071067811865476


def _gelu_pre(m):
    return (_SQRT_HALF * m) * (1.0 + jax.lax.erf(m))


def _fused_body(x_ref, wl_ref, g_ref, bt_ref, W1_ref, b1_ref, W2a_ref,
                W2b_ref, b2_ref, W3_ref, b3_ref, w4d_ref, o_ref):
    X = x_ref[0]
    NT = N_TOKENS * T_BLK
    Xf = X.reshape(NT, N_C)

    mu = jnp.mean(Xf, axis=-1, keepdims=True)
    xc = Xf - mu
    var = jnp.mean(xc * xc, axis=-1, keepdims=True)
    ln = xc / jnp.sqrt(var + 1e-5) * g_ref[...] + bt_ref[...]
    u = _gelu_pre(jnp.dot(ln, W1_ref[...], preferred_element_type=jnp.float32)
                  + b1_ref[...])
    u3 = u.reshape(N_TOKENS, T_BLK, N_C)
    glob = jnp.sum(u3[:, :, N_C // 2:], axis=0) / float(N_TOKENS)
    gpart = jnp.dot(glob, W2b_ref[...],
                    preferred_element_type=jnp.float32) + b2_ref[...]
    lpart = jnp.dot(u3[:, :, :N_C // 2].reshape(NT, N_C // 2), W2a_ref[...],
                    preferred_element_type=jnp.float32)
    h2 = _gelu_pre(lpart.reshape(N_TOKENS, T_BLK, N_C // 2)
                   + gpart[None]).reshape(NT, N_C // 2)
    h3 = _gelu_pre(jnp.dot(h2, W3_ref[...],
                           preferred_element_type=jnp.float32) + b3_ref[...])
    d = jnp.sum(h3 * w4d_ref[...], axis=-1).reshape(N_TOKENS, T_BLK)
    dt = d.T

    k = jax.lax.bitcast_convert_type(dt, jnp.int32)
    key = k ^ ((k >> 31) & jnp.int32(0x7FFFFFFF))
    ukey = key ^ jnp.int32(-2147483648)
    active = jnp.ones((T_BLK, N_TOKENS), jnp.int32)
    bottom = jnp.zeros((T_BLK, N_TOKENS), jnp.int32)
    need = jnp.full((T_BLK, 1), N_BOT, jnp.int32)
    for bit in range(31, -1, -1):
        bitv = (ukey >> bit) & 1
        zeros = active * (bitv ^ 1)
        nz = jnp.sum(zeros, axis=1, keepdims=True)
        go_zero = nz >= need
        bottom = jnp.where(go_zero, bottom, bottom + zeros)
        need = jnp.where(go_zero, need, need - nz)
        active = jnp.where(go_zero, zeros, active * bitv)
    idx = jax.lax.broadcasted_iota(jnp.int32, (T_BLK, N_TOKENS), 1)
    for bit in range(9, -1, -1):
        bitv = (idx >> bit) & 1
        ones = active * bitv
        n1 = jnp.sum(ones, axis=1, keepdims=True)
        go_one = n1 >= need
        bottom = jnp.where(go_one, bottom, bottom + ones)
        need = jnp.where(go_one, need, need - n1)
        active = jnp.where(go_one, ones, active * (bitv ^ 1))
    bottom = bottom + active * jnp.where(need > 0, 1, 0)

    bot_f = bottom.astype(jnp.float32).T[:, :, None]
    Sr = jnp.sum(X, axis=0)
    botr = jnp.sum(X * bot_f, axis=0)
    wl2 = wl_ref[...]
    S = Sr * wl2
    pooled = S / float(N_TOKENS)
    add2 = pooled + (S * (1.0 + wl2) - botr * wl2
                     - float(N_BOT) * pooled * wl2) / float(N_TOKENS - N_BOT)
    o_ref[0] = X + add2[None]


def kernel(x, wl, ln_g, ln_b, W1, b1, W2, b2, W3, b3, W4, b4):
    wl2 = wl.reshape(1, N_C)
    g2 = ln_g.reshape(1, N_C)
    bt2 = ln_b.reshape(1, N_C)
    W1s = W1 * _SQRT_HALF
    b1r = (b1 * _SQRT_HALF).reshape(1, N_C)
    W2a = W2[:N_C // 2] * _SQRT_HALF
    W2b = W2[N_C // 2:] * _SQRT_HALF
    b2r = (b2 * _SQRT_HALF).reshape(1, N_C // 2)
    W3s = W3 * _SQRT_HALF
    b3r = (b3 * _SQRT_HALF).reshape(1, N_C // 4)
    w4d = (W4[:, 0] - W4[:, 1]).reshape(1, N_C // 4)

    grid = (N_B, N_T // T_BLK)
    blk = pl.BlockSpec((1, N_TOKENS, T_BLK, N_C), lambda b, t: (b, 0, t, 0))
    small = lambda s: pl.BlockSpec(s, lambda b, t: (0,) * len(s))
    return pl.pallas_call(
        _fused_body,
        grid=grid,
        in_specs=[
            blk,
            small((1, N_C)), small((1, N_C)), small((1, N_C)),
            small((N_C, N_C)), small((1, N_C)),
            small((N_C // 2, N_C // 2)), small((N_C // 2, N_C // 2)),
            small((1, N_C // 2)),
            small((N_C // 2, N_C // 4)), small((1, N_C // 4)),
            small((1, N_C // 4)),
        ],
        out_specs=blk,
        out_shape=jax.ShapeDtypeStruct((N_B, N_TOKENS, N_T, N_C), jnp.float32),
        compiler_params=pltpu.CompilerParams(
            dimension_semantics=("parallel", "parallel")),
    )(x, wl2, g2, bt2, W1s, b1r, W2a, W2b, b2r, W3s, b3r, w4d)


# speedup vs baseline: 1.0398x; 1.0398x over previous
import jax
import jax.numpy as jnp
from jax.experimental import pallas as pl
from jax.experimental.pallas import tpu as pltpu

N_B, N_TOKENS, N_T, N_C = 8, 675, 16, 256
N_BOT = 135
T_BLK = 8


_SQRT_HALF = 0.7071067811865476


def _gelu_pre(m):
    return (_SQRT_HALF * m) * (1.0 + jax.lax.erf(m))


def _fused_body(x_ref, wl_ref, W1_ref, b1_ref, W2a_ref,
                W2b_ref, b2_ref, W3_ref, b3_ref, w4d_ref, o_ref):
    X = x_ref[0]
    NT = N_TOKENS * T_BLK
    Xf = X.reshape(NT, N_C)

    mu = jnp.mean(Xf, axis=-1, keepdims=True)
    xc = Xf - mu
    var = jnp.mean(xc * xc, axis=-1, keepdims=True)
    ln = xc * jax.lax.rsqrt(var + 1e-5)
    u = _gelu_pre(jnp.dot(ln, W1_ref[...], preferred_element_type=jnp.float32)
                  + b1_ref[...])
    u3 = u.reshape(N_TOKENS, T_BLK, N_C)
    glob = jnp.sum(u3[:, :, N_C // 2:], axis=0) / float(N_TOKENS)
    gpart = jnp.dot(glob, W2b_ref[...],
                    preferred_element_type=jnp.float32) + b2_ref[...]
    lpart = jnp.dot(u3[:, :, :N_C // 2].reshape(NT, N_C // 2), W2a_ref[...],
                    preferred_element_type=jnp.float32)
    h2 = _gelu_pre(lpart.reshape(N_TOKENS, T_BLK, N_C // 2)
                   + gpart[None]).reshape(NT, N_C // 2)
    h3 = _gelu_pre(jnp.dot(h2, W3_ref[...],
                           preferred_element_type=jnp.float32) + b3_ref[...])
    d = jnp.sum(h3 * w4d_ref[...], axis=-1).reshape(N_TOKENS, T_BLK)
    dt = d.T

    k = jax.lax.bitcast_convert_type(dt, jnp.int32)
    key = k ^ ((k >> 31) & jnp.int32(0x7FFFFFFF))
    ukey = key ^ jnp.int32(-2147483648)
    active = jnp.ones((T_BLK, N_TOKENS), jnp.int32)
    bottom = jnp.zeros((T_BLK, N_TOKENS), jnp.int32)
    need = jnp.full((T_BLK, 1), N_BOT, jnp.int32)
    for bit in range(31, -1, -1):
        bitv = (ukey >> bit) & 1
        zeros = active * (bitv ^ 1)
        nz = jnp.sum(zeros, axis=1, keepdims=True)
        go_zero = nz >= need
        bottom = jnp.where(go_zero, bottom, bottom + zeros)
        need = jnp.where(go_zero, need, need - nz)
        active = jnp.where(go_zero, zeros, active * bitv)
    idx = jax.lax.broadcasted_iota(jnp.int32, (T_BLK, N_TOKENS), 1)
    for bit in range(9, -1, -1):
        bitv = (idx >> bit) & 1
        ones = active * bitv
        n1 = jnp.sum(ones, axis=1, keepdims=True)
        go_one = n1 >= need
        bottom = jnp.where(go_one, bottom, bottom + ones)
        need = jnp.where(go_one, need, need - n1)
        active = jnp.where(go_one, ones, active * (bitv ^ 1))
    bottom = bottom + active * jnp.where(need > 0, 1, 0)

    bot_f = bottom.astype(jnp.float32).T[:, :, None]
    Sr = jnp.sum(X, axis=0)
    botr = jnp.sum(X * bot_f, axis=0)
    wl2 = wl_ref[...]
    S = Sr * wl2
    pooled = S / float(N_TOKENS)
    add2 = pooled + (S * (1.0 + wl2) - botr * wl2
                     - float(N_BOT) * pooled * wl2) / float(N_TOKENS - N_BOT)
    o_ref[0] = X + add2[None]


def kernel(x, wl, ln_g, ln_b, W1, b1, W2, b2, W3, b3, W4, b4):
    wl2 = wl.reshape(1, N_C)
    W1s = (ln_g[:, None] * W1) * _SQRT_HALF
    b1r = ((ln_b @ W1 + b1) * _SQRT_HALF).reshape(1, N_C)
    W2a = W2[:N_C // 2] * _SQRT_HALF
    W2b = W2[N_C // 2:] * _SQRT_HALF
    b2r = (b2 * _SQRT_HALF).reshape(1, N_C // 2)
    W3s = W3 * _SQRT_HALF
    b3r = (b3 * _SQRT_HALF).reshape(1, N_C // 4)
    w4d = (W4[:, 0] - W4[:, 1]).reshape(1, N_C // 4)

    grid = (N_B, N_T // T_BLK)
    blk = pl.BlockSpec((1, N_TOKENS, T_BLK, N_C), lambda b, t: (b, 0, t, 0))
    small = lambda s: pl.BlockSpec(s, lambda b, t: (0,) * len(s))
    return pl.pallas_call(
        _fused_body,
        grid=grid,
        in_specs=[
            blk,
            small((1, N_C)),
            small((N_C, N_C)), small((1, N_C)),
            small((N_C // 2, N_C // 2)), small((N_C // 2, N_C // 2)),
            small((1, N_C // 2)),
            small((N_C // 2, N_C // 4)), small((1, N_C // 4)),
            small((1, N_C // 4)),
        ],
        out_specs=blk,
        out_shape=jax.ShapeDtypeStruct((N_B, N_TOKENS, N_T, N_C), jnp.float32),
        compiler_params=pltpu.CompilerParams(
            dimension_semantics=("parallel", "parallel")),
    )(x, wl2, W1s, b1r, W2a, W2b, b2r, W3s, b3r, w4d)


# speedup vs baseline: 1.0975x; 1.0555x over previous
import jax
import jax.numpy as jnp
from jax.experimental import pallas as pl
from jax.experimental.pallas import tpu as pltpu

N_B, N_TOKENS, N_T, N_C = 8, 675, 16, 256
N_BOT = 135
T_BLK = 8


_SQRT_HALF = 0.7071067811865476


def _gelu_pre(m):
    return (_SQRT_HALF * m) * (1.0 + jax.lax.erf(m))


def _fused_body(x_ref, wl_ref, W1_ref, b1_ref, W2_ref,
                b2_ref, W3_ref, b3_ref, w4d_ref, o_ref):
    X = x_ref[0]
    NT = N_TOKENS * T_BLK
    Xf = X.reshape(NT, N_C)

    mu = jnp.mean(Xf, axis=-1, keepdims=True)
    xc = Xf - mu
    var = jnp.mean(xc * xc, axis=-1, keepdims=True)
    ln = xc * jax.lax.rsqrt(var + 1e-5)
    u = _gelu_pre(jnp.dot(ln, W1_ref[...], preferred_element_type=jnp.float32)
                  + b1_ref[...])
    u3 = u.reshape(N_TOKENS, T_BLK, N_C)
    glob = jnp.sum(u3[:, :, N_C // 2:], axis=0) / float(N_TOKENS)
    H = jnp.concatenate(
        [u3[:, :, :N_C // 2],
         jnp.broadcast_to(glob[None], (N_TOKENS, T_BLK, N_C // 2))], axis=-1)
    h2 = _gelu_pre(jnp.dot(H.reshape(NT, N_C), W2_ref[...],
                           preferred_element_type=jnp.float32) + b2_ref[...])
    h3 = _gelu_pre(jnp.dot(h2, W3_ref[...],
                           preferred_element_type=jnp.float32) + b3_ref[...])
    d = jnp.sum(h3 * w4d_ref[...], axis=-1).reshape(N_TOKENS, T_BLK)
    dt = d.T

    k = jax.lax.bitcast_convert_type(dt, jnp.int32)
    key = k ^ ((k >> 31) & jnp.int32(0x7FFFFFFF))
    ukey = key ^ jnp.int32(-2147483648)
    active = jnp.ones((T_BLK, N_TOKENS), jnp.int32)
    bottom = jnp.zeros((T_BLK, N_TOKENS), jnp.int32)
    need = jnp.full((T_BLK, 1), N_BOT, jnp.int32)
    for bit in range(31, -1, -1):
        bitv = (ukey >> bit) & 1
        zeros = active * (bitv ^ 1)
        nz = jnp.sum(zeros, axis=1, keepdims=True)
        go_zero = nz >= need
        bottom = jnp.where(go_zero, bottom, bottom + zeros)
        need = jnp.where(go_zero, need, need - nz)
        active = jnp.where(go_zero, zeros, active * bitv)
    idx = jax.lax.broadcasted_iota(jnp.int32, (T_BLK, N_TOKENS), 1)
    for bit in range(9, -1, -1):
        bitv = (idx >> bit) & 1
        ones = active * bitv
        n1 = jnp.sum(ones, axis=1, keepdims=True)
        go_one = n1 >= need
        bottom = jnp.where(go_one, bottom, bottom + ones)
        need = jnp.where(go_one, need, need - n1)
        active = jnp.where(go_one, ones, active * (bitv ^ 1))
    bottom = bottom + active * jnp.where(need > 0, 1, 0)

    bot_f = bottom.astype(jnp.float32).T[:, :, None]
    Sr = jnp.sum(X, axis=0)
    botr = jnp.sum(X * bot_f, axis=0)
    wl2 = wl_ref[...]
    S = Sr * wl2
    pooled = S / float(N_TOKENS)
    add2 = pooled + (S * (1.0 + wl2) - botr * wl2
                     - float(N_BOT) * pooled * wl2) / float(N_TOKENS - N_BOT)
    o_ref[0] = X + add2[None]


def kernel(x, wl, ln_g, ln_b, W1, b1, W2, b2, W3, b3, W4, b4):
    wl2 = wl.reshape(1, N_C)
    W1s = (ln_g[:, None] * W1) * _SQRT_HALF
    b1r = ((ln_b @ W1 + b1) * _SQRT_HALF).reshape(1, N_C)
    W2s = W2 * _SQRT_HALF
    b2r = (b2 * _SQRT_HALF).reshape(1, N_C // 2)
    W3s = W3 * _SQRT_HALF
    b3r = (b3 * _SQRT_HALF).reshape(1, N_C // 4)
    w4d = (W4[:, 0] - W4[:, 1]).reshape(1, N_C // 4)

    grid = (N_B, N_T // T_BLK)
    blk = pl.BlockSpec((1, N_TOKENS, T_BLK, N_C), lambda b, t: (b, 0, t, 0))
    small = lambda s: pl.BlockSpec(s, lambda b, t: (0,) * len(s))
    return pl.pallas_call(
        _fused_body,
        grid=grid,
        in_specs=[
            blk,
            small((1, N_C)),
            small((N_C, N_C)), small((1, N_C)),
            small((N_C, N_C // 2)), small((1, N_C // 2)),
            small((N_C // 2, N_C // 4)), small((1, N_C // 4)),
            small((1, N_C // 4)),
        ],
        out_specs=blk,
        out_shape=jax.ShapeDtypeStruct((N_B, N_TOKENS, N_T, N_C), jnp.float32),
        compiler_params=pltpu.CompilerParams(
            dimension_semantics=("parallel", "parallel")),
    )(x, wl2, W1s, b1r, W2s, b2r, W3s, b3r, w4d)
